# baseline (speedup 1.0000x reference)
import dataclasses
import functools

import jax
import jax.numpy as jnp
from jax import lax
from jax.experimental import pallas as pl
from jax.experimental.pallas import tpu as pltpu
from jax.experimental.pallas import tpu_sc as plsc


def kernel(text_strings, text_lengths, embedding_table):
    B, L = text_strings.shape
    V, D = embedding_table.shape
    NC, NS = 2, 16
    NW = NC * NS
    RPW = B // NW
    C = 8
    G = 4
    NG = RPW // G

    TB = 16384
    v = text_strings
    ids_flat = (2 * ((v // (2 * TB)) * TB + v % TB)
                + (v // TB) % 2).reshape(B * L)
    lengths = text_lengths.astype(jnp.int32)
    mesh = plsc.VectorSubcoreMesh(core_axis_name="c", subcore_axis_name="s")
    cp = pltpu.CompilerParams()
    if "use_tc_tiling_on_sc" in pltpu.CompilerParams.__dataclass_fields__:
        cp = dataclasses.replace(cp, use_tc_tiling_on_sc=False)
    if "needs_layout_passes" in pltpu.CompilerParams.__dataclass_fields__:
        cp = dataclasses.replace(cp, needs_layout_passes=False)

    @functools.partial(
        pl.kernel,
        compiler_params=cp,
        out_type=jax.ShapeDtypeStruct((B * D,), jnp.float32),
        mesh=mesh,
        scratch_types=[
            pltpu.VMEM((2, G * L), jnp.int32),
            pltpu.VMEM((2, G, L, D), jnp.float32),
            pltpu.VMEM((2, G * D), jnp.float32),
            pltpu.VMEM((RPW,), jnp.int32),
            pltpu.SemaphoreType.DMA,
            pltpu.SemaphoreType.DMA,
            pltpu.SemaphoreType.DMA,
            pltpu.SemaphoreType.DMA,
            pltpu.SemaphoreType.DMA,
            pltpu.SemaphoreType.DMA,
        ],
    )
    def bow_kernel(ids_hbm, len_hbm, tab_hbm, out_hbm,
                   ids_v, rows_v, out_v, lens_v,
                   gsem0, gsem1, isem0, isem1, osem0, osem1):
        wid = lax.axis_index("s") * NC + lax.axis_index("c")
        base = wid * RPW
        pltpu.sync_copy(len_hbm.at[pl.ds(base, RPW)], lens_v)
        gsems = (gsem0, gsem1)
        isems = (isem0, isem1)
        osems = (osem0, osem1)

        def nchunks(n):
            return lax.div(n + (C - 1), C)

        def length_at(idx):
            blk = lax.div(idx, 16) * 16
            lens16 = lens_v[pl.ds(blk, 16)]
            pos = jnp.full((16,), idx - blk, jnp.int32)
            sel = jnp.where(lax.iota(jnp.int32, 16) == pos, lens16,
                            jnp.zeros((16,), jnp.int32))
            return jnp.sum(sel)

        def issue_ids(buf, grp):
            pltpu.async_copy(
                ids_hbm.at[pl.ds((base + grp * G) * L, G * L)],
                ids_v.at[buf], isems[buf])

        def ids_drain(buf):
            pltpu.make_async_copy(
                ids_hbm.at[pl.ds(0, G * L)], ids_v.at[buf],
                isems[buf]).wait()

        def fire_group(buf, grp):
            row0 = grp * G
            ids_b = ids_v.at[buf]
            ids_drain(buf)
            for r in range(G):
                nc = nchunks(length_at(row0 + r))

                @pl.loop(0, nc)
                def _(c):
                    pltpu.async_copy(
                        tab_hbm.at[ids_b.at[pl.ds(r * L + c * C, C)]],
                        rows_v.at[buf, r, pl.ds(c * C, C)],
                        gsems[buf])

        def drain_and_compute_group(buf, grp, issue_ids_grp):
            row0 = grp * G
            tot = nchunks(length_at(row0))
            for r in range(1, G):
                tot = tot + nchunks(length_at(row0 + r))

            @pl.loop(0, tot)
            def _(c):
                pltpu.make_async_copy(
                    tab_hbm.at[pl.ds(0, C)],
                    rows_v.at[buf, 0, pl.ds(0, C)],
                    gsems[buf]).wait()

            @pl.when(issue_ids_grp < NG)
            def _():
                issue_ids(buf, issue_ids_grp)

            @pl.when(grp >= 2)
            def _():
                pltpu.make_async_copy(
                    out_hbm.at[pl.ds(0, G * D)], out_v.at[buf],
                    osems[buf]).wait()

            zero = jnp.zeros((16,), jnp.float32)
            for r in range(G):
                n = length_at(row0 + r)
                nv = jnp.full((16,), n, jnp.int32)
                nf = nv.astype(jnp.float32)
                inv = jnp.where(nv > 0, 1.0 / jnp.maximum(nf, 1.0), zero)

                def body(p, accs):
                    l = 2 * p
                    a = tuple(
                        accs[j] + rows_v[buf, r, l, pl.ds(16 * j, 16)]
                        for j in range(4)
                    )
                    return tuple(
                        a[j] + rows_v[buf, r, l + 1, pl.ds(16 * j, 16)]
                        for j in range(4)
                    )

                accs = lax.fori_loop(0, lax.div(n, 2), body,
                                     (zero, zero, zero, zero), unroll=False)
                lt = jnp.maximum(n - 1, 0)
                odd = jnp.full((16,), lax.rem(n, 2), jnp.int32) > 0
                accs = tuple(
                    jnp.where(odd,
                              accs[j] + rows_v[buf, r, lt, pl.ds(16 * j, 16)],
                              accs[j])
                    for j in range(4)
                )
                for j in range(4):
                    out_v[buf, pl.ds(r * D + 16 * j, 16)] = accs[j] * inv
            pltpu.async_copy(
                out_v.at[buf],
                out_hbm.at[pl.ds((base + row0) * D, G * D)], osems[buf])

        issue_ids(0, 0)
        fire_group(0, 0)
        issue_ids(1, 1)

        @pl.loop(0, NG, step=2)
        def _(k):
            fire_group(1, k + 1)
            drain_and_compute_group(0, k, k + 2)

            @pl.when(k + 2 < NG)
            def _():
                fire_group(0, k + 2)

            drain_and_compute_group(1, k + 1, k + 3)

        for buf in range(2):
            pltpu.make_async_copy(
                out_hbm.at[pl.ds(0, G * D)], out_v.at[buf],
                osems[buf]).wait()

    GRID = pl.cdiv(V, 2 * TB)
    ROWS = GRID * TB
    LASTB = (V - 1) // TB

    def _transpose_body(a_ref, b_ref, o_ref):
        o_ref[:, 0:D] = a_ref[...].T
        o_ref[:, D:2 * D] = b_ref[...].T

    table_wide = pl.pallas_call(
        _transpose_body,
        grid=(GRID,),
        in_specs=[
            pl.BlockSpec((D, TB), lambda i: (0, 2 * i)),
            pl.BlockSpec((D, TB), lambda i: (0, jnp.minimum(2 * i + 1, LASTB))),
        ],
        out_specs=pl.BlockSpec((TB, 2 * D), lambda i: (i, 0)),
        out_shape=jax.ShapeDtypeStruct((ROWS, 2 * D), jnp.float32),
    )(embedding_table.T, embedding_table.T)
    table_rm = table_wide.reshape(2 * ROWS, D)

    out_flat = bow_kernel(ids_flat, lengths, table_rm)
    return out_flat.reshape(B, D)
